# baseline (speedup 1.0000x reference)
import jax
import jax.numpy as jnp
import numpy as np
from jax import lax
from jax.experimental import pallas as pl
from jax.experimental.pallas import tpu as pltpu

_N_BINS = 15
_N = 100000
_C = 1000
_SUB = 8
_NSTREAMS = 5
_TROWS = _C // _SUB
_GRID_I = _TROWS // _NSTREAMS
_S = 25600
_NCHUNK = -(-_N // _S)
_S8 = _S // 8

_BOUNDS = [float(b) for b in
           np.linspace(0.0, 1.0, _N_BINS + 1).astype(np.float32)[:_N_BINS]]


def _ece_body(p0, p1, p2, p3, p4, labels_ref, out_ref, m_ref, i_ref):
    c = pl.program_id(0)
    i = pl.program_id(1)
    refs = (p0, p1, p2, p3, p4)

    def upd(x, tile_row):
        m = m_ref[...]
        p = x > m
        m_ref[...] = jnp.where(p, x, m)
        i_ref[...] = jnp.where(p, tile_row, i_ref[...])

    @pl.when(i == 0)
    def _init():
        m_ref[...] = p0[...]
        i_ref[...] = jnp.zeros_like(i_ref)
        for k in range(1, _NSTREAMS):
            upd(refs[k][...], k)

    @pl.when(i > 0)
    def _upd():
        for k in range(_NSTREAMS):
            upd(refs[k][...], _NSTREAMS * i + k)

    @pl.when(i == _GRID_I - 1)
    def _fin():
        m = m_ref[...]
        conf = jnp.max(m, axis=0, keepdims=True)
        sub = lax.broadcasted_iota(jnp.int32, m.shape, 0)
        cls = i_ref[...] * _SUB + sub
        pred = jnp.min(jnp.where(m == conf, cls, _C), axis=0, keepdims=True)

        correct = (pred == labels_ref[0]).astype(jnp.float32)

        pos = lax.broadcasted_iota(jnp.int32, conf.shape, 1)
        valid = (c * _S + pos) < _N

        nbelow = jnp.zeros_like(conf, dtype=jnp.int32)
        for b in _BOUNDS:
            nbelow = nbelow + (conf > b).astype(jnp.int32)
        bin_idx = nbelow - 1

        zero = jnp.zeros_like(conf)
        rows = []
        for j in range(_N_BINS):
            sel = (bin_idx == j) & valid
            cnt = jnp.sum(jnp.where(sel, 1.0, 0.0))
            s_cf = jnp.sum(jnp.where(sel, conf, zero))
            s_co = jnp.sum(jnp.where(sel, correct, zero))
            rows.append((cnt, s_cf, s_co))
        part = jnp.stack([jnp.stack([r[q] for r in rows])
                          for q in range(3)]).reshape(3, _N_BINS)

        @pl.when(c == 0)
        def _first():
            out_ref[0:3, 0:_N_BINS] = part

        @pl.when(c > 0)
        def _acc():
            out_ref[0:3, 0:_N_BINS] += part


@jax.jit
def _ece_pallas(pt, labels3):
    def pspec(k):
        return pl.BlockSpec((_SUB, _S),
                            lambda c, i, k=k: (_NSTREAMS * i + k, c))

    out = pl.pallas_call(
        _ece_body,
        grid=(_NCHUNK, _GRID_I),
        in_specs=[pspec(k) for k in range(_NSTREAMS)]
                 + [pl.BlockSpec((1, 1, _S), lambda c, i: (c, 0, 0))],
        out_specs=pl.BlockSpec((8, 128), lambda c, i: (0, 0)),
        out_shape=jax.ShapeDtypeStruct((8, 128), jnp.float32),
        scratch_shapes=[pltpu.VMEM((_SUB, _S), jnp.float32),
                        pltpu.VMEM((_SUB, _S), jnp.int32)],
        compiler_params=pltpu.CompilerParams(
            dimension_semantics=("arbitrary", "arbitrary"),
        ),
    )(*([pt] * _NSTREAMS), labels3)
    return out


def kernel(probs, labels, mode):
    del mode
    pt = probs.T
    labels3 = jnp.pad(labels, (0, _NCHUNK * _S - _N)).reshape(_NCHUNK, 1, _S)
    out = _ece_pallas(pt, labels3)
    count = out[0, 0:_N_BINS]
    s_conf = out[1, 0:_N_BINS]
    s_corr = out[2, 0:_N_BINS]
    ece = jnp.sum(jnp.abs(s_conf - s_corr)).reshape(1)
    return (ece, s_corr, count)


# speedup vs baseline: 1.3823x; 1.3823x over previous
import jax
import jax.numpy as jnp
import numpy as np
from jax import lax
from jax.experimental import pallas as pl
from jax.experimental.pallas import tpu as pltpu

_N_BINS = 15
_N = 100000
_C = 1000
_SUB = 8
_NSTREAMS = 5
_TROWS = _C // _SUB
_GRID = _TROWS // _NSTREAMS

_ALL_BOUNDS = [float(b) for b in
               np.linspace(0.0, 1.0, _N_BINS + 1).astype(np.float32)]


def _ece_body(p0, p1, p2, p3, p4, labels_ref, out_ref, m_ref, i_ref):
    i = pl.program_id(0)
    refs = (p0, p1, p2, p3, p4)

    def upd(x, tile_row):
        m = m_ref[...]
        p = x > m
        m_ref[...] = jnp.where(p, x, m)
        i_ref[...] = jnp.where(p, tile_row, i_ref[...])

    @pl.when(i == 0)
    def _init():
        m_ref[...] = p0[...]
        i_ref[...] = jnp.zeros_like(i_ref)
        for k in range(1, _NSTREAMS):
            upd(refs[k][...], k)

    @pl.when(i > 0)
    def _upd():
        for k in range(_NSTREAMS):
            upd(refs[k][...], _NSTREAMS * i + k)

    @pl.when(i == _GRID - 1)
    def _fin():
        m = m_ref[...]
        conf = jnp.max(m, axis=0, keepdims=True)
        sub = lax.broadcasted_iota(jnp.int32, m.shape, 0)
        cls = i_ref[...] * _SUB + sub
        pred = jnp.min(jnp.where(m == conf, cls, _C), axis=0, keepdims=True)
        correct = (pred == labels_ref[...]).astype(jnp.float32)

        conf8 = jnp.broadcast_to(conf, m.shape)
        corr8 = jnp.broadcast_to(correct, m.shape)
        one = jnp.ones_like(conf8)
        vmat = jnp.where(sub == 0, one,
                         jnp.where(sub == 1, conf8,
                                   jnp.where(sub == 2, corr8, 0.0)))

        for h in range(2):
            omat = jnp.zeros_like(conf8)
            for r in range(8):
                b = _ALL_BOUNDS[8 * h + r]
                omat = jnp.where((sub == r) & (conf8 > b), one, omat)
            cum = lax.dot_general(
                omat, vmat, (((1,), (1,)), ((), ())),
                preferred_element_type=jnp.float32)
            out_ref[0:8, 8 * h:8 * h + 8] = cum


@jax.jit
def _ece_pallas(pt, labels2):
    def pspec(k):
        return pl.BlockSpec((_SUB, _N), lambda i, k=k: (_NSTREAMS * i + k, 0))

    out = pl.pallas_call(
        _ece_body,
        grid=(_GRID,),
        in_specs=[pspec(k) for k in range(_NSTREAMS)]
                 + [pl.BlockSpec((1, _N), lambda i: (0, 0))],
        out_specs=pl.BlockSpec((8, 128), lambda i: (0, 0)),
        out_shape=jax.ShapeDtypeStruct((8, 128), jnp.float32),
        scratch_shapes=[pltpu.VMEM((_SUB, _N), jnp.float32),
                        pltpu.VMEM((_SUB, _N), jnp.int32)],
        compiler_params=pltpu.CompilerParams(
            dimension_semantics=("arbitrary",),
        ),
    )(*([pt] * _NSTREAMS), labels2)
    return out


def kernel(probs, labels, mode):
    del mode
    pt = probs.T
    labels2 = labels.reshape(1, _N)
    out = _ece_pallas(pt, labels2)
    cum_cnt = jnp.concatenate([out[0:8, 0], out[0:8, 8]])
    cum_cf = jnp.concatenate([out[0:8, 1], out[0:8, 9]])
    cum_co = jnp.concatenate([out[0:8, 2], out[0:8, 10]])
    count = cum_cnt[:_N_BINS] - cum_cnt[1:]
    s_conf = cum_cf[:_N_BINS] - cum_cf[1:]
    s_corr = cum_co[:_N_BINS] - cum_co[1:]
    ece = jnp.sum(jnp.abs(s_conf - s_corr)).reshape(1)
    return (ece, s_corr, count)
